# baseline (speedup 1.0000x reference)
import functools

import jax
import jax.numpy as jnp
from jax import lax
from jax.experimental import pallas as pl
from jax.experimental.pallas import tpu as pltpu
from jax.experimental.pallas import tpu_sc as plsc

BS, SEQ1, D = 4, 8193, 768
TOT = BS * D
_INFO = plsc.get_sparse_core_info()
NC, NS, L = _INFO.num_cores, _INFO.num_subcores, _INFO.num_lanes
NW = NC * NS
PER_W = TOT // NW
CHUNKS_PER_B = NW // BS

_mesh = plsc.VectorSubcoreMesh(core_axis_name="c", subcore_axis_name="s")


@functools.partial(
    pl.kernel,
    mesh=_mesh,
    out_type=jax.ShapeDtypeStruct((TOT,), jnp.float32),
    scratch_types=[
        pltpu.VMEM((PER_W,), jnp.int32),
        pltpu.VMEM((PER_W,), jnp.int32),
        pltpu.VMEM((PER_W,), jnp.float32),
        pltpu.SemaphoreType.DMA,
    ],
)
def _sc_gather(tok_hbm, idx_hbm, out_hbm, idx_v, flat_v, val_v, sem):
    wid = lax.axis_index("s") * NC + lax.axis_index("c")
    b = wid // CHUNKS_PER_B
    jbase = (wid % CHUNKS_PER_B) * PER_W
    pltpu.sync_copy(idx_hbm.at[pl.ds(jbase, PER_W)], idx_v)
    lane = lax.iota(jnp.int32, L)
    for i in range(PER_W // L):
        iv = idx_v[pl.ds(i * L, L)]
        flat_v[pl.ds(i * L, L)] = b * (SEQ1 * D) + (iv + 1) * D + jbase + i * L + lane
    pltpu.async_copy(tok_hbm.at[flat_v], val_v, sem).wait()
    pltpu.sync_copy(val_v, out_hbm.at[pl.ds(wid * PER_W, PER_W)])


def _tc_dense_body(emb_ref, cls_ref, win_ref, wout_ref, out_ref):
    q = lax.dot_general(
        cls_ref[...], win_ref[...], (((1,), (1,)), ((), ())),
        precision=lax.Precision.HIGHEST, preferred_element_type=jnp.float32)
    comb = jnp.concatenate([emb_ref[...], q], axis=1)
    out_ref[...] = jnp.tanh(lax.dot_general(
        comb, wout_ref[...], (((1,), (1,)), ((), ())),
        precision=lax.Precision.HIGHEST, preferred_element_type=jnp.float32))


_tc_dense = pl.pallas_call(
    _tc_dense_body,
    out_shape=jax.ShapeDtypeStruct((BS, D), jnp.float32),
)


def kernel(output_tokens_from_bert, cls_output, embrace_idx, W_in, W_out):
    tok_flat = output_tokens_from_bert.reshape(-1)
    emb = _sc_gather(tok_flat, embrace_idx).reshape(BS, D)
    return _tc_dense(emb, cls_output, W_in, W_out)


# speedup vs baseline: 15.2573x; 15.2573x over previous
import functools

import jax
import jax.numpy as jnp
from jax import lax
from jax.experimental import pallas as pl
from jax.experimental.pallas import tpu as pltpu
from jax.experimental.pallas import tpu_sc as plsc

BS, SEQ1, D = 4, 8193, 768
TOT = BS * D
_INFO = plsc.get_sparse_core_info()
NC, NS, L = _INFO.num_cores, _INFO.num_subcores, _INFO.num_lanes
NW = NC * NS
PER_W = TOT // NW
CHUNKS_PER_B = NW // BS

_mesh = plsc.VectorSubcoreMesh(core_axis_name="c", subcore_axis_name="s")


@functools.partial(
    pl.kernel,
    mesh=_mesh,
    out_type=jax.ShapeDtypeStruct((TOT,), jnp.float32),
    scratch_types=[
        pltpu.VMEM((PER_W,), jnp.int32),
        pltpu.VMEM((PER_W,), jnp.int32),
        pltpu.VMEM((PER_W, D), jnp.float32),
        pltpu.VMEM((PER_W,), jnp.float32),
        pltpu.SemaphoreType.DMA,
    ],
    compiler_params=pltpu.CompilerParams(
        use_tc_tiling_on_sc=True, needs_layout_passes=False),
)
def _sc_gather(tok_hbm, idx_hbm, out_hbm, idx_v, row_v, rows_buf, val_v, sem):
    wid = lax.axis_index("s") * NC + lax.axis_index("c")
    b = wid // CHUNKS_PER_B
    jbase = (wid % CHUNKS_PER_B) * PER_W
    pltpu.sync_copy(idx_hbm.at[pl.ds(jbase, PER_W)], idx_v)
    lane = lax.iota(jnp.int32, L)
    for i in range(PER_W // L):
        row_v[pl.ds(i * L, L)] = idx_v[pl.ds(i * L, L)] + 1
    pltpu.async_copy(tok_hbm.at[b].at[row_v], rows_buf, sem).wait()
    for i in range(PER_W // L):
        k = i * L + lane
        val_v[pl.ds(i * L, L)] = plsc.load_gather(rows_buf, [k, jbase + k])
    pltpu.sync_copy(val_v, out_hbm.at[pl.ds(wid * PER_W, PER_W)])


def _tc_dense_body(emb_ref, cls_ref, win_ref, wout_ref, out_ref):
    q = lax.dot_general(
        cls_ref[...], win_ref[...], (((1,), (1,)), ((), ())),
        precision=lax.Precision.HIGHEST, preferred_element_type=jnp.float32)
    comb = jnp.concatenate([emb_ref[...], q], axis=1)
    out_ref[...] = jnp.tanh(lax.dot_general(
        comb, wout_ref[...], (((1,), (1,)), ((), ())),
        precision=lax.Precision.HIGHEST, preferred_element_type=jnp.float32))


_tc_dense = pl.pallas_call(
    _tc_dense_body,
    out_shape=jax.ShapeDtypeStruct((BS, D), jnp.float32),
)


def kernel(output_tokens_from_bert, cls_output, embrace_idx, W_in, W_out):
    emb = _sc_gather(output_tokens_from_bert, embrace_idx).reshape(BS, D)
    return _tc_dense(emb, cls_output, W_in, W_out)


# speedup vs baseline: 44.5012x; 2.9167x over previous
import functools

import jax
import jax.numpy as jnp
from jax import lax
from jax.experimental import pallas as pl
from jax.experimental.pallas import tpu as pltpu
from jax.experimental.pallas import tpu_sc as plsc

BS, SEQ1, D = 4, 8193, 768
TOT = BS * D
NT = D // 128
_INFO = plsc.get_sparse_core_info()
NC, NS, L = _INFO.num_cores, _INFO.num_subcores, _INFO.num_lanes
NW = NC * NS
PER_W = TOT // NW
JW = D // NW

_mesh = plsc.VectorSubcoreMesh(core_axis_name="c", subcore_axis_name="s", num_cores=1)


@functools.partial(
    pl.kernel,
    mesh=_mesh,
    out_type=jax.ShapeDtypeStruct((TOT,), jnp.float32),
    scratch_types=[
        pltpu.VMEM((JW,), jnp.int32),
        pltpu.VMEM((JW,), jnp.int32),
        pltpu.VMEM((JW, BS, D), jnp.float32),
        pltpu.VMEM((PER_W,), jnp.float32),
        pltpu.SemaphoreType.DMA,
    ],
    compiler_params=pltpu.CompilerParams(
        use_tc_tiling_on_sc=True, needs_layout_passes=False),
)
def _sc_gather(tok_hbm, idx_hbm, out_hbm, idx_v, row_v, slab_v, val_v, sem):
    sid = lax.axis_index("s")
    lane = lax.iota(jnp.int32, L)
    for half in range(2):
        wid = sid + half * NS
        jbase = wid * JW
        pltpu.sync_copy(idx_hbm.at[pl.ds(jbase, JW)], idx_v)
        for st in (0, JW - L):
            row_v[pl.ds(st, L)] = idx_v[pl.ds(st, L)] + 1
        pltpu.async_copy(tok_hbm.at[row_v], slab_v, sem).wait()
        for i in range(PER_W // L):
            e = i * L + lane
            b_v = e // JW
            k_v = e - b_v * JW
            val_v[pl.ds(i * L, L)] = plsc.load_gather(
                slab_v, [k_v, b_v, jbase + k_v])
        for b in range(BS):
            pltpu.sync_copy(val_v.at[pl.ds(b * JW, JW)],
                            out_hbm.at[pl.ds(b * D + jbase, JW)])


def _tc_q_body(cls_ref, win_ref, woutb_ref, h2_ref):
    q = lax.dot_general(
        cls_ref[...], win_ref[...], (((1,), (1,)), ((), ())),
        precision=lax.Precision.HIGHEST, preferred_element_type=jnp.float32)
    h2_ref[...] = lax.dot_general(
        q, woutb_ref[...], (((1,), (1,)), ((), ())),
        precision=lax.Precision.HIGHEST, preferred_element_type=jnp.float32)


_tc_q = pl.pallas_call(
    _tc_q_body,
    grid=(1,),
    out_shape=jax.ShapeDtypeStruct((BS, D), jnp.float32),
    in_specs=[
        pl.BlockSpec((BS, D), lambda i: (0, 0)),
        pl.BlockSpec((D, D), lambda i: (0, 0)),
        pl.BlockSpec((D, D), lambda i: (0, 1)),
    ],
    out_specs=pl.BlockSpec((BS, D), lambda i: (0, 0)),
)


def _tc_out_body(emb_ref, h2_ref, wouta_ref, out_ref):
    emb = jnp.concatenate(
        [emb_ref[pl.ds(b * D, D)].reshape(1, D) for b in range(BS)], axis=0)
    out_ref[...] = jnp.tanh(h2_ref[...] + lax.dot_general(
        emb, wouta_ref[...], (((1,), (1,)), ((), ())),
        precision=lax.Precision.HIGHEST, preferred_element_type=jnp.float32))


_tc_out = pl.pallas_call(
    _tc_out_body,
    grid=(NT,),
    out_shape=jax.ShapeDtypeStruct((BS, D), jnp.float32),
    in_specs=[
        pl.BlockSpec((TOT,), lambda i: (0,)),
        pl.BlockSpec((BS, 128), lambda i: (0, i)),
        pl.BlockSpec((128, D), lambda i: (i, 0)),
    ],
    out_specs=pl.BlockSpec((BS, 128), lambda i: (0, i)),
)


def kernel(output_tokens_from_bert, cls_output, embrace_idx, W_in, W_out):
    tok_t = jnp.swapaxes(output_tokens_from_bert, 0, 1)
    emb_flat = _sc_gather(tok_t, embrace_idx)
    h2 = _tc_q(cls_output, W_in, W_out)
    return _tc_out(emb_flat, h2, W_out)


# speedup vs baseline: 53.2178x; 1.1959x over previous
import functools

import jax
import jax.numpy as jnp
from jax import lax
from jax.experimental import pallas as pl
from jax.experimental.pallas import tpu as pltpu
from jax.experimental.pallas import tpu_sc as plsc

BS, SEQ1, D = 4, 8193, 768
TOT = BS * D
NT = D // 128
_INFO = plsc.get_sparse_core_info()
NC, NS, L = _INFO.num_cores, _INFO.num_subcores, _INFO.num_lanes
NW = NC * NS
PER_W = TOT // NW
JW = D // NW

_mesh = plsc.VectorSubcoreMesh(core_axis_name="c", subcore_axis_name="s")


@functools.partial(
    pl.kernel,
    mesh=_mesh,
    out_type=jax.ShapeDtypeStruct((TOT,), jnp.float32),
    scratch_types=[
        pltpu.VMEM((JW,), jnp.int32),
        pltpu.VMEM((JW,), jnp.int32),
        pltpu.VMEM((JW, BS, 256), jnp.float32),
        pltpu.VMEM((PER_W,), jnp.float32),
        pltpu.SemaphoreType.DMA,
    ],
    compiler_params=pltpu.CompilerParams(
        use_tc_tiling_on_sc=True, needs_layout_passes=False),
)
def _sc_gather(tok_hbm, idx_hbm, out_hbm, idx_v, row_v, slab_v, val_v, sem):
    wid = lax.axis_index("s") * NC + lax.axis_index("c")
    jbase = wid * JW
    ab = pl.multiple_of(jnp.minimum((jbase >> 7) << 7, D - 256), 128)
    pltpu.sync_copy(idx_hbm.at[pl.ds(jbase, JW)], idx_v)
    lane = lax.iota(jnp.int32, L)
    for st in (0, JW - L):
        row_v[pl.ds(st, L)] = idx_v[pl.ds(st, L)] + 1
    pltpu.async_copy(tok_hbm.at[row_v, :, pl.ds(ab, 256)], slab_v, sem).wait()
    for i in range(PER_W // L):
        e = i * L + lane
        b_v = e // JW
        k_v = e - b_v * JW
        val_v[pl.ds(i * L, L)] = plsc.load_gather(
            slab_v, [k_v, b_v, jbase + k_v - ab])
    for b in range(BS):
        pltpu.sync_copy(val_v.at[pl.ds(b * JW, JW)],
                        out_hbm.at[pl.ds(b * D + jbase, JW)])


def _tc_q_body(cls_ref, win_ref, woutb_ref, h2_ref):
    q = lax.dot_general(
        cls_ref[...], win_ref[...], (((1,), (1,)), ((), ())),
        precision=lax.Precision.HIGHEST, preferred_element_type=jnp.float32)
    h2_ref[...] = lax.dot_general(
        q, woutb_ref[...], (((1,), (1,)), ((), ())),
        precision=lax.Precision.HIGHEST, preferred_element_type=jnp.float32)


_tc_q = pl.pallas_call(
    _tc_q_body,
    grid=(1,),
    out_shape=jax.ShapeDtypeStruct((BS, D), jnp.float32),
    in_specs=[
        pl.BlockSpec((BS, D), lambda i: (0, 0)),
        pl.BlockSpec((D, D), lambda i: (0, 0)),
        pl.BlockSpec((D, D), lambda i: (0, 1)),
    ],
    out_specs=pl.BlockSpec((BS, D), lambda i: (0, 0)),
)


def _tc_out_body(emb_ref, h2_ref, wouta_ref, out_ref):
    emb = jnp.concatenate(
        [emb_ref[pl.ds(b * D, D)].reshape(1, D) for b in range(BS)], axis=0)
    out_ref[...] = jnp.tanh(h2_ref[...] + lax.dot_general(
        emb, wouta_ref[...], (((1,), (1,)), ((), ())),
        precision=lax.Precision.HIGHEST, preferred_element_type=jnp.float32))


_tc_out = pl.pallas_call(
    _tc_out_body,
    grid=(1,),
    out_shape=jax.ShapeDtypeStruct((BS, D), jnp.float32),
    in_specs=[
        pl.BlockSpec((TOT,), lambda i: (0,)),
        pl.BlockSpec((BS, D), lambda i: (0, 0)),
        pl.BlockSpec((D, D), lambda i: (0, 0)),
    ],
    out_specs=pl.BlockSpec((BS, D), lambda i: (0, 0)),
)


def kernel(output_tokens_from_bert, cls_output, embrace_idx, W_in, W_out):
    tok_t = jnp.swapaxes(output_tokens_from_bert, 0, 1)
    emb_flat = _sc_gather(tok_t, embrace_idx)
    h2 = _tc_q(cls_output, W_in, W_out)
    return _tc_out(emb_flat, h2, W_out)
